# Initial kernel scaffold; baseline (speedup 1.0000x reference)
import jax
import jax.numpy as jnp
from jax.experimental import pallas as pl


def kernel(inputs, W, pos_enc):
    raise NotImplementedError("write your pallas kernel here")


# speedup vs baseline: 2.3579x; 2.3579x over previous
import functools

import jax
import jax.numpy as jnp
from jax import lax
from jax.experimental import pallas as pl
from jax.experimental.pallas import tpu as pltpu
from jax.experimental.pallas import tpu_sc as plsc

_NC = 2
_NS = 16
_NW = _NC * _NS
_L = 16

_CHUNK = 128
_NBUF = 5


@functools.partial(jax.jit, static_argnums=(3, 4))
def _sc_embed(idx1d, W, pos2, S, D):
    R = idx1d.shape[0]
    cpw = R // (_NW * _CHUNK)
    nouter = cpw // _NBUF
    rows_per_worker = cpw * _CHUNK

    mesh = plsc.VectorSubcoreMesh(
        core_axis_name="c", subcore_axis_name="s",
        num_cores=_NC, num_subcores=_NS)

    @functools.partial(
        pl.kernel,
        out_type=jax.ShapeDtypeStruct((R, D), jnp.float32),
        mesh=mesh,
        scratch_types=[
            pltpu.VMEM((cpw * _CHUNK,), jnp.int32),
            pltpu.VMEM((2 * S, D), jnp.float32),
            pltpu.VMEM((_NBUF, _CHUNK, D), jnp.float32),
        ] + [pltpu.SemaphoreType.DMA] * _NBUF,
        compiler_params=pltpu.CompilerParams(use_tc_tiling_on_sc=False),
    )
    def body(w_hbm, idx_hbm, pos2_hbm, out_hbm, idx_v, pos_v, rows_v, *gsems):
        wid = lax.axis_index("s") * _NC + lax.axis_index("c")
        base_row = wid * rows_per_worker

        pltpu.sync_copy(
            idx_hbm.at[pl.ds(wid * rows_per_worker, rows_per_worker)], idx_v)
        pltpu.sync_copy(pos2_hbm, pos_v)

        for b in range(_NBUF):
            pltpu.make_async_copy(
                w_hbm.at[idx_v.at[pl.ds(b * _CHUNK, _CHUNK)]],
                rows_v.at[b], gsems[b]).start()

        @pl.loop(0, nouter)
        def _outer(g0):
            for b in range(_NBUF):
                g = g0 * _NBUF + b
                pltpu.make_async_copy(
                    w_hbm.at[idx_v.at[pl.ds(b * _CHUNK, _CHUNK)]],
                    rows_v.at[b], gsems[b]).wait()

                s0 = lax.rem(g * _CHUNK, S)

                @pl.loop(0, _CHUNK)
                def _radd(r):
                    for j in range(D // _L):
                        p = pos_v[s0 + r, pl.ds(j * _L, _L)]
                        plsc.addupdate(rows_v.at[b, r, pl.ds(j * _L, _L)], p)

                pltpu.sync_copy(
                    rows_v.at[b],
                    out_hbm.at[pl.ds(base_row + g * _CHUNK, _CHUNK)])

                nxt = g + _NBUF

                @pl.when(nxt < cpw)
                def _():
                    pltpu.make_async_copy(
                        w_hbm.at[idx_v.at[pl.ds(nxt * _CHUNK, _CHUNK)]],
                        rows_v.at[b], gsems[b]).start()

    return body(W, idx1d, pos2)


def kernel(inputs, W, pos_enc):
    B, S = inputs.shape
    V, D = W.shape
    R = B * S
    assert R % (_NW * _CHUNK) == 0
    assert (R // _NW) % S == 0
    assert (R // (_NW * _CHUNK)) % _NBUF == 0
    assert D % _L == 0
    assert S - 2 + _CHUNK <= 2 * S

    idx1d = inputs.reshape(R)
    pos2 = jnp.concatenate([pos_enc, pos_enc], axis=0)
    out = _sc_embed(idx1d, W, pos2, S, D)
    return out.reshape(B, S, D)


# speedup vs baseline: 2.5142x; 1.0663x over previous
import functools

import jax
import jax.numpy as jnp
from jax import lax
from jax.experimental import pallas as pl
from jax.experimental.pallas import tpu as pltpu
from jax.experimental.pallas import tpu_sc as plsc

_NC = 2
_NS = 16
_NW = _NC * _NS
_L = 16

_CHUNK = 128
_NBUF = 10


@functools.partial(jax.jit, static_argnums=(3, 4))
def _sc_embed(idx1d, W, pos2, S, D):
    R = idx1d.shape[0]
    cpw = R // (_NW * _CHUNK)
    nouter = cpw // _NBUF
    rows_per_worker = cpw * _CHUNK

    mesh = plsc.VectorSubcoreMesh(
        core_axis_name="c", subcore_axis_name="s",
        num_cores=_NC, num_subcores=_NS)

    @functools.partial(
        pl.kernel,
        out_type=jax.ShapeDtypeStruct((R, D), jnp.float32),
        mesh=mesh,
        scratch_types=[
            pltpu.VMEM((cpw * _CHUNK,), jnp.int32),
            pltpu.VMEM((2 * S, D), jnp.float32),
            pltpu.VMEM((_NBUF, _CHUNK, D), jnp.float32),
        ] + [pltpu.SemaphoreType.DMA] * (2 * _NBUF),
        compiler_params=pltpu.CompilerParams(use_tc_tiling_on_sc=False),
    )
    def body(w_hbm, idx_hbm, pos2_hbm, out_hbm, idx_v, pos_v, rows_v, *sems):
        gsems = sems[:_NBUF]
        osems = sems[_NBUF:]
        wid = lax.axis_index("s") * _NC + lax.axis_index("c")
        base_row = wid * rows_per_worker

        def gather(chunk, b):
            pltpu.make_async_copy(
                w_hbm.at[idx_v.at[pl.ds(chunk * _CHUNK, _CHUNK)]],
                rows_v.at[b], gsems[b]).start()

        def gather_wait(b):
            pltpu.make_async_copy(
                w_hbm.at[idx_v.at[pl.ds(b * _CHUNK, _CHUNK)]],
                rows_v.at[b], gsems[b]).wait()

        def store(chunk, b):
            pltpu.make_async_copy(
                rows_v.at[b],
                out_hbm.at[pl.ds(base_row + chunk * _CHUNK, _CHUNK)],
                osems[b]).start()

        def store_wait(b):
            pltpu.make_async_copy(
                rows_v.at[b], out_hbm.at[pl.ds(base_row, _CHUNK)],
                osems[b]).wait()

        pltpu.sync_copy(
            idx_hbm.at[pl.ds(wid * rows_per_worker, rows_per_worker)], idx_v)
        pltpu.sync_copy(pos2_hbm, pos_v)

        for b in range(_NBUF):
            gather(b, b)

        @pl.loop(0, nouter)
        def _outer(g0):
            for b in range(_NBUF):
                g = g0 * _NBUF + b
                gather_wait(b)

                s0 = lax.rem(g * _CHUNK, S)

                @pl.loop(0, _CHUNK, unroll=8)
                def _radd(r):
                    for j in range(D // _L):
                        p = pos_v[s0 + r, pl.ds(j * _L, _L)]
                        plsc.addupdate(rows_v.at[b, r, pl.ds(j * _L, _L)], p)

                store(g, b)

                pb = (b - 1) % _NBUF
                pg = g - 1

                @pl.when((pg >= 0) & (pg + _NBUF < cpw))
                def _():
                    store_wait(pb)
                    gather(pg + _NBUF, pb)

        for b in range(_NBUF):
            store_wait(b)

    return body(W, idx1d, pos2)


def kernel(inputs, W, pos_enc):
    B, S = inputs.shape
    V, D = W.shape
    R = B * S
    assert R % (_NW * _CHUNK) == 0
    assert (R // _NW) % S == 0
    assert (R // (_NW * _CHUNK)) % _NBUF == 0
    assert D % _L == 0
    assert S - 2 + _CHUNK <= 2 * S

    idx1d = inputs.reshape(R)
    pos2 = jnp.concatenate([pos_enc, pos_enc], axis=0)
    out = _sc_embed(idx1d, W, pos2, S, D)
    return out.reshape(B, S, D)


# speedup vs baseline: 3.2257x; 1.2830x over previous
import functools

import jax
import jax.numpy as jnp
from jax import lax
from jax.experimental import pallas as pl
from jax.experimental.pallas import tpu as pltpu
from jax.experimental.pallas import tpu_sc as plsc

_NC = 2
_NS = 16
_NW = _NC * _NS
_L = 16

_NBUF = 8


@functools.partial(jax.jit, static_argnums=())
def _sc_embed(inputs, W, pos_enc):
    B, S = inputs.shape
    D = W.shape[1]
    spw = B // _NW
    wins = [(o, min(128, S - o)) for o in range(0, S, 128)]

    mesh = plsc.VectorSubcoreMesh(
        core_axis_name="c", subcore_axis_name="s",
        num_cores=_NC, num_subcores=_NS)

    @functools.partial(
        pl.kernel,
        out_type=jax.ShapeDtypeStruct((B, S, D), jnp.float32),
        mesh=mesh,
        scratch_types=[
            pltpu.VMEM((spw, S), jnp.int32),
            pltpu.VMEM((S, D), jnp.float32),
            pltpu.VMEM((_NBUF, S, D), jnp.float32),
        ] + [pltpu.SemaphoreType.DMA] * (2 * _NBUF),
        compiler_params=pltpu.CompilerParams(use_tc_tiling_on_sc=False),
    )
    def body(w_hbm, idx_hbm, pos_hbm, out_hbm, idx_v, pos_v, rows_v, *sems):
        gsems = sems[:_NBUF]
        osems = sems[_NBUF:]
        wid = lax.axis_index("s") * _NC + lax.axis_index("c")
        base_seq = wid * spw

        def gather(seq, b):
            for o, n in wins:
                pltpu.make_async_copy(
                    w_hbm.at[idx_v.at[seq, pl.ds(o, n)]],
                    rows_v.at[b, pl.ds(o, n)], gsems[b]).start()

        def gather_wait(b):
            for o, n in wins:
                pltpu.make_async_copy(
                    w_hbm.at[idx_v.at[0, pl.ds(o, n)]],
                    rows_v.at[b, pl.ds(o, n)], gsems[b]).wait()

        def store(seq, b):
            pltpu.make_async_copy(
                rows_v.at[b], out_hbm.at[base_seq + seq], osems[b]).start()

        def store_wait(b):
            pltpu.make_async_copy(
                rows_v.at[b], out_hbm.at[base_seq], osems[b]).wait()

        pltpu.sync_copy(idx_hbm.at[pl.ds(wid * spw, spw)], idx_v)
        pltpu.sync_copy(pos_hbm, pos_v)

        for b in range(_NBUF):
            gather(b, b)

        @pl.loop(0, spw // _NBUF)
        def _outer(g0):
            for b in range(_NBUF):
                g = g0 * _NBUF + b
                gather_wait(b)

                @pl.loop(0, S, unroll=8)
                def _radd(r):
                    for j in range(D // _L):
                        p = pos_v[r, pl.ds(j * _L, _L)]
                        plsc.addupdate(rows_v.at[b, r, pl.ds(j * _L, _L)], p)

                store(g, b)

                pb = (b - 1) % _NBUF
                pg = g - 1

                @pl.when((pg >= 0) & (pg + _NBUF < spw))
                def _():
                    store_wait(pb)
                    gather(pg + _NBUF, pb)

        for b in range(_NBUF):
            store_wait(b)

    return body(W, inputs, pos_enc)


def kernel(inputs, W, pos_enc):
    B, S = inputs.shape
    V, D = W.shape
    assert B % _NW == 0
    assert (B // _NW) % _NBUF == 0
    assert D % _L == 0
    assert S % 8 == 0
    return _sc_embed(inputs, W, pos_enc)
